# baseline (speedup 1.0000x reference)
import functools

import jax
import jax.numpy as jnp
from jax import lax
from jax.experimental import pallas as pl
from jax.experimental.pallas import tpu as pltpu
from jax.experimental.pallas import tpu_sc as plsc

_B, _H, _W = 1024, 32, 32
_HW = _H * _W
_NCOLORS = 10
_D = 64
_N = _B * _HW

_LANES = 16
_SUB = 128
_GPC = 2
_CHUNK = _SUB * _GPC


def _table_body(ct_ref, rt_ref, colt_ref, w_ref, b_ref, out_ref):
    pid = lax.broadcasted_iota(jnp.int32, (_HW, _W), 0)
    j = lax.broadcasted_iota(jnp.int32, (_HW, _W), 1)
    row_oh = (pid // _W == j).astype(jnp.float32)
    col_oh = (pid % _W == j).astype(jnp.float32)
    row_part = jnp.dot(row_oh, rt_ref[...],
                       preferred_element_type=jnp.float32,
                       precision=lax.Precision.HIGHEST)
    col_part = jnp.dot(col_oh, colt_ref[...],
                       preferred_element_type=jnp.float32,
                       precision=lax.Precision.HIGHEST)
    pc = jnp.concatenate([row_part, col_part], axis=1)
    pe = lax.dot_general(pc, w_ref[...], (((1,), (1,)), ((), ())),
                         preferred_element_type=jnp.float32,
                         precision=lax.Precision.HIGHEST)
    pe = pe + b_ref[...][None, :]
    out_ref[...] = ct_ref[...][:, None, :] + pe[None, :, :]


def _build_sum_table(color_table, row_table, col_table, proj_w, proj_b):
    out = pl.pallas_call(
        _table_body,
        out_shape=jax.ShapeDtypeStruct((_NCOLORS, _HW, _D), jnp.float32),
    )(color_table, row_table, col_table, proj_w, proj_b)
    return out.reshape(_NCOLORS * _HW, _D)


def _make_sc_kernel():
    info = plsc.get_sparse_core_info()
    nc, ns = info.num_cores, info.num_subcores
    nw = nc * ns
    groups_per_w = (_N // _SUB) // nw
    chunks_per_w = groups_per_w // _GPC

    mesh = plsc.VectorSubcoreMesh(core_axis_name="c", subcore_axis_name="s")

    @functools.partial(
        pl.kernel,
        mesh=mesh,
        compiler_params=pltpu.CompilerParams(use_tc_tiling_on_sc=False),
        out_type=jax.ShapeDtypeStruct((_B, _HW, 128), jnp.float32),
        scratch_types=[
            pltpu.VMEM((3 * _GPC, _SUB), jnp.int32),
            pltpu.VMEM((3, _CHUNK, _D), jnp.float32),
            pltpu.VMEM_SHARED((_NCOLORS * _HW, _D), jnp.float32),
            pltpu.SemaphoreType.DMA,
            pltpu.SemaphoreType.DMA,
            pltpu.SemaphoreType.DMA,
            pltpu.SemaphoreType.DMA,
            pltpu.SemaphoreType.DMA,
            pltpu.SemaphoreType.DMA,
        ],
    )
    def sc_kernel(grid_hbm, table_hbm, out_hbm, idx_v, rows_v, tbl_s,
                  gsem0, gsem1, gsem2, osem0, osem1, osem2):
        gsem = (gsem0, gsem1, gsem2)
        osem = (osem0, osem1, osem2)
        wid = lax.axis_index("s") * nc + lax.axis_index("c")
        w_group_base = wid * groups_per_w

        @pl.when(lax.axis_index("s") == 0)
        def _():
            pltpu.sync_copy(table_hbm, tbl_s)
        plsc.subcore_barrier()

        def gather_descs(k, b):
            return [
                pltpu.make_async_copy(
                    tbl_s.at[idx_v.at[b * _GPC + g]],
                    rows_v.at[b, pl.ds(g * _SUB, _SUB)],
                    gsem[b],
                )
                for g in range(_GPC)
            ]

        def out_desc(k, b):
            row_base = (w_group_base + k * _GPC) * _SUB
            return pltpu.make_async_copy(
                rows_v.at[b],
                out_hbm.at[row_base // _HW, pl.ds(lax.rem(row_base, _HW),
                                                 _CHUNK), pl.ds(0, _D)],
                osem[b])

        def stage(k, b):
            group_base = w_group_base + k * _GPC
            pltpu.sync_copy(
                grid_hbm.at[pl.ds(group_base, _GPC)],
                idx_v.at[pl.ds(b * _GPC, _GPC)])
            for g in range(_GPC):
                phase = lax.rem(group_base + g, _HW // _SUB) * _SUB
                for t in range(_SUB // _LANES):
                    pvec = phase + t * _LANES + lax.iota(jnp.int32, _LANES)
                    r = b * _GPC + g
                    sl = idx_v[r, pl.ds(t * _LANES, _LANES)]
                    idx_v[r, pl.ds(t * _LANES, _LANES)] = sl * _HW + pvec

        def stage_and_fire(k, b):
            stage(k, b)
            for d in gather_descs(k, b):
                d.start()

        def generic(k, b):
            b2 = (b + 2) % 3
            for d in gather_descs(k, b):
                d.wait()
            out_desc(k, b).start()
            stage(k + 2, b2)
            out_desc(k - 1, b2).wait()
            for d in gather_descs(k + 2, b2):
                d.start()

        stage_and_fire(0, 0)
        stage_and_fire(1, 1)

        for d in gather_descs(0, 0):
            d.wait()
        out_desc(0, 0).start()
        stage(2, 2)
        for d in gather_descs(2, 2):
            d.start()

        def triple_body(m, carry):
            for half in range(3):
                k = 3 * m + 1 + half
                generic(k, (1 + half) % 3)
            return carry

        lax.fori_loop(0, (chunks_per_w - 4) // 3, triple_body, None)
        for k in range(3 * ((chunks_per_w - 4) // 3) + 1, chunks_per_w - 2):
            generic(k, k % 3)

        for k in (chunks_per_w - 2, chunks_per_w - 1):
            for d in gather_descs(k, k % 3):
                d.wait()
            out_desc(k, k % 3).start()
        for k in (chunks_per_w - 3, chunks_per_w - 2, chunks_per_w - 1):
            out_desc(k, k % 3).wait()

    return sc_kernel


def kernel(grid, color_table, row_table, col_table, proj_w, proj_b):
    sum_table = _build_sum_table(color_table, row_table, col_table,
                                 proj_w, proj_b)
    grid2d = grid.reshape(_N // _SUB, _SUB)
    sc_kernel = _make_sc_kernel()
    out = sc_kernel(grid2d, sum_table)
    return out[:, :, :_D]


# speedup vs baseline: 1.0352x; 1.0352x over previous
import functools

import jax
import jax.numpy as jnp
from jax import lax
from jax.experimental import pallas as pl
from jax.experimental.pallas import tpu as pltpu
from jax.experimental.pallas import tpu_sc as plsc

_B, _H, _W = 1024, 32, 32
_HW = _H * _W
_NCOLORS = 10
_D = 64
_N = _B * _HW

_LANES = 16
_SUB = 128
_GPC = 4
_CHUNK = _SUB * _GPC


def _table_body(ct_ref, rt_ref, colt_ref, w_ref, b_ref, out_ref):
    pid = lax.broadcasted_iota(jnp.int32, (_HW, _W), 0)
    j = lax.broadcasted_iota(jnp.int32, (_HW, _W), 1)
    row_oh = (pid // _W == j).astype(jnp.float32)
    col_oh = (pid % _W == j).astype(jnp.float32)
    row_part = jnp.dot(row_oh, rt_ref[...],
                       preferred_element_type=jnp.float32,
                       precision=lax.Precision.HIGHEST)
    col_part = jnp.dot(col_oh, colt_ref[...],
                       preferred_element_type=jnp.float32,
                       precision=lax.Precision.HIGHEST)
    pc = jnp.concatenate([row_part, col_part], axis=1)
    pe = lax.dot_general(pc, w_ref[...], (((1,), (1,)), ((), ())),
                         preferred_element_type=jnp.float32,
                         precision=lax.Precision.HIGHEST)
    pe = pe + b_ref[...][None, :]
    out_ref[...] = ct_ref[...][:, None, :] + pe[None, :, :]


def _build_sum_table(color_table, row_table, col_table, proj_w, proj_b):
    out = pl.pallas_call(
        _table_body,
        out_shape=jax.ShapeDtypeStruct((_NCOLORS, _HW, _D), jnp.float32),
    )(color_table, row_table, col_table, proj_w, proj_b)
    return out.reshape(_NCOLORS * _HW, _D)


def _make_sc_kernel():
    info = plsc.get_sparse_core_info()
    nc, ns = info.num_cores, info.num_subcores
    nw = nc * ns
    groups_per_w = (_N // _SUB) // nw
    chunks_per_w = groups_per_w // _GPC

    mesh = plsc.VectorSubcoreMesh(core_axis_name="c", subcore_axis_name="s")

    @functools.partial(
        pl.kernel,
        mesh=mesh,
        compiler_params=pltpu.CompilerParams(use_tc_tiling_on_sc=False),
        out_type=jax.ShapeDtypeStruct((_B, _HW, 128), jnp.float32),
        scratch_types=[
            pltpu.VMEM((2 * _GPC, _SUB), jnp.int32),
            pltpu.VMEM((2, _CHUNK, _D), jnp.float32),
            pltpu.VMEM_SHARED((_NCOLORS * _HW, _D), jnp.float32),
            pltpu.SemaphoreType.DMA,
            pltpu.SemaphoreType.DMA,
            pltpu.SemaphoreType.DMA,
            pltpu.SemaphoreType.DMA,
        ],
    )
    def sc_kernel(grid_hbm, table_hbm, out_hbm, idx_v, rows_v, tbl_s,
                  gsem0, gsem1, osem0, osem1):
        gsem = (gsem0, gsem1)
        osem = (osem0, osem1)
        wid = lax.axis_index("s") * nc + lax.axis_index("c")
        w_group_base = wid * groups_per_w

        @pl.when(lax.axis_index("s") == 0)
        def _():
            pltpu.sync_copy(table_hbm, tbl_s)
        plsc.subcore_barrier()

        def gather_descs(k, b):
            return [
                pltpu.make_async_copy(
                    tbl_s.at[idx_v.at[b * _GPC + g]],
                    rows_v.at[b, pl.ds(g * _SUB, _SUB)],
                    gsem[b],
                )
                for g in range(_GPC)
            ]

        def out_desc(k, b):
            row_base = (w_group_base + k * _GPC) * _SUB
            return pltpu.make_async_copy(
                rows_v.at[b],
                out_hbm.at[row_base // _HW, pl.ds(lax.rem(row_base, _HW),
                                                 _CHUNK), pl.ds(0, _D)],
                osem[b])

        def stage(k, b):
            group_base = w_group_base + k * _GPC
            pltpu.sync_copy(
                grid_hbm.at[pl.ds(group_base, _GPC)],
                idx_v.at[pl.ds(b * _GPC, _GPC)])
            for g in range(_GPC):
                phase = lax.rem(group_base + g, _HW // _SUB) * _SUB
                for t in range(_SUB // _LANES):
                    pvec = phase + t * _LANES + lax.iota(jnp.int32, _LANES)
                    r = b * _GPC + g
                    sl = idx_v[r, pl.ds(t * _LANES, _LANES)]
                    idx_v[r, pl.ds(t * _LANES, _LANES)] = sl * _HW + pvec

        def stage_and_fire(k, b):
            stage(k, b)
            for d in gather_descs(k, b):
                d.start()

        stage_and_fire(0, 0)
        stage_and_fire(1, 1)

        def pair_body(m, carry):
            for half in range(2):
                k = 2 * m + half
                b = half
                for d in gather_descs(k, b):
                    d.wait()
                out_desc(k, b).start()
                stage(k + 2, b)
                out_desc(k, b).wait()
                for d in gather_descs(k + 2, b):
                    d.start()
            return carry

        lax.fori_loop(0, chunks_per_w // 2 - 1, pair_body, None)

        for half in range(2):
            k = chunks_per_w - 2 + half
            for d in gather_descs(k, half):
                d.wait()
            out_desc(k, half).start()
        out_desc(chunks_per_w - 2, 0).wait()
        out_desc(chunks_per_w - 1, 1).wait()

    return sc_kernel


def kernel(grid, color_table, row_table, col_table, proj_w, proj_b):
    sum_table = _build_sum_table(color_table, row_table, col_table,
                                 proj_w, proj_b)
    grid2d = grid.reshape(_N // _SUB, _SUB)
    sc_kernel = _make_sc_kernel()
    out = sc_kernel(grid2d, sum_table)
    return out[:, :, :_D]
